# initial kernel scaffold (unmeasured)
import jax
import jax.numpy as jnp
from jax import lax
from jax.experimental import pallas as pl
from jax.experimental.pallas import tpu as pltpu


def kernel(
    x,
):
    def body(*refs):
        pass

    out_shape = jax.ShapeDtypeStruct(..., jnp.float32)
    return pl.pallas_call(body, out_shape=out_shape)(...)


# baseline (device time: 105819 ns/iter reference)
import jax
import jax.numpy as jnp
from jax import lax
from jax.experimental import pallas as pl
from jax.experimental.pallas import tpu as pltpu


def kernel(x):
    m, n = x.shape

    def body(x_ref, out_ref, send_sem, recv_sem):
        mx = lax.axis_index("x")
        my = lax.axis_index("y")
        yp = (mx, 1 - my)

        barrier = pltpu.get_barrier_semaphore()
        pl.semaphore_signal(
            barrier, inc=1, device_id=yp, device_id_type=pl.DeviceIdType.MESH
        )
        pl.semaphore_wait(barrier, 1)

        out_ref[pl.ds(my * m, m), :] = x_ref[...].astype(out_ref.dtype)

        send = pltpu.make_async_remote_copy(
            src_ref=out_ref.at[pl.ds(my * m, m), :],
            dst_ref=out_ref.at[pl.ds(my * m, m), :],
            send_sem=send_sem,
            recv_sem=recv_sem,
            device_id=yp,
            device_id_type=pl.DeviceIdType.MESH,
        )
        send.start()
        send.wait_send()

        recv = pltpu.make_async_remote_copy(
            src_ref=out_ref.at[pl.ds((1 - my) * m, m), :],
            dst_ref=out_ref.at[pl.ds((1 - my) * m, m), :],
            send_sem=send_sem,
            recv_sem=recv_sem,
            device_id=yp,
            device_id_type=pl.DeviceIdType.MESH,
        )
        recv.wait_recv()

    out_shape = jax.ShapeDtypeStruct((2 * m, n), jnp.bfloat16)
    return pl.pallas_call(
        body,
        out_shape=out_shape,
        in_specs=[pl.BlockSpec(memory_space=pltpu.VMEM)],
        out_specs=pl.BlockSpec(memory_space=pltpu.VMEM),
        scratch_shapes=[
            pltpu.SemaphoreType.DMA,
            pltpu.SemaphoreType.DMA,
        ],
        compiler_params=pltpu.CompilerParams(collective_id=0),
    )(x)


# device time: 68706 ns/iter; 1.5402x vs baseline; 1.5402x over previous
import jax
import jax.numpy as jnp
from jax import lax
from jax.experimental import pallas as pl
from jax.experimental.pallas import tpu as pltpu

K = 8


def kernel(x):
    m, n = x.shape
    h = m // 2
    c = h // K

    def body(x_ref, out_ref, y_send_sems, y_recv_sems, x_send_sems, x_recv_sems):
        mx = lax.axis_index("x")
        my = lax.axis_index("y")
        yp = (mx, 1 - my)
        xp = (1 - mx, my)

        barrier = pltpu.get_barrier_semaphore()
        for nbr in (yp, xp):
            pl.semaphore_signal(
                barrier, inc=1, device_id=nbr, device_id_type=pl.DeviceIdType.MESH
            )
        pl.semaphore_wait(barrier, 2)

        def copy(row, send_sem, recv_sem, dev):
            return pltpu.make_async_remote_copy(
                src_ref=out_ref.at[pl.ds(row, c), :],
                dst_ref=out_ref.at[pl.ds(row, c), :],
                send_sem=send_sem,
                recv_sem=recv_sem,
                device_id=dev,
                device_id_type=pl.DeviceIdType.MESH,
            )

        out_ref[pl.ds(my * m + mx * h, h), :] = x_ref[pl.ds(mx * h, h), :].astype(
            out_ref.dtype
        )

        y_sends = []
        for k in range(K):
            s = copy(my * m + mx * h + k * c, y_send_sems.at[k], y_recv_sems.at[k], yp)
            s.start()
            y_sends.append(s)

        out_ref[pl.ds(my * m + (1 - mx) * h, h), :] = x_ref[
            pl.ds((1 - mx) * h, h), :
        ].astype(out_ref.dtype)

        x_sends = []
        for k in range(K):
            row = (1 - my) * m + mx * h + k * c
            copy(row, y_send_sems.at[k], y_recv_sems.at[k], yp).wait_recv()
            f = copy(row, x_send_sems.at[k], x_recv_sems.at[k], xp)
            f.start()
            x_sends.append(f)

        for k in range(K):
            row = (1 - my) * m + (1 - mx) * h + k * c
            copy(row, x_send_sems.at[k], x_recv_sems.at[k], xp).wait_recv()

        for s in y_sends:
            s.wait_send()
        for s in x_sends:
            s.wait_send()

    out_shape = jax.ShapeDtypeStruct((2 * m, n), jnp.bfloat16)
    return pl.pallas_call(
        body,
        out_shape=out_shape,
        in_specs=[pl.BlockSpec(memory_space=pltpu.VMEM)],
        out_specs=pl.BlockSpec(memory_space=pltpu.VMEM),
        scratch_shapes=[
            pltpu.SemaphoreType.DMA((K,)),
            pltpu.SemaphoreType.DMA((K,)),
            pltpu.SemaphoreType.DMA((K,)),
            pltpu.SemaphoreType.DMA((K,)),
        ],
        compiler_params=pltpu.CompilerParams(collective_id=0),
    )(x)


# device time: 65762 ns/iter; 1.6091x vs baseline; 1.0448x over previous
import jax
import jax.numpy as jnp
from jax import lax
from jax.experimental import pallas as pl
from jax.experimental.pallas import tpu as pltpu

K = 16


def kernel(x):
    m, n = x.shape
    h = m // 2
    c = h // K

    def body(x_ref, out_ref, y_send_sems, y_recv_sems, x_send_sems, x_recv_sems):
        mx = lax.axis_index("x")
        my = lax.axis_index("y")
        yp = (mx, 1 - my)
        xp = (1 - mx, my)

        barrier = pltpu.get_barrier_semaphore()
        for nbr in (yp, xp):
            pl.semaphore_signal(
                barrier, inc=1, device_id=nbr, device_id_type=pl.DeviceIdType.MESH
            )
        pl.semaphore_wait(barrier, 2)

        def copy(row, send_sem, recv_sem, dev):
            return pltpu.make_async_remote_copy(
                src_ref=out_ref.at[pl.ds(row, c), :],
                dst_ref=out_ref.at[pl.ds(row, c), :],
                send_sem=send_sem,
                recv_sem=recv_sem,
                device_id=dev,
                device_id_type=pl.DeviceIdType.MESH,
            )

        y_sends = []
        for k in range(K):
            row = my * m + mx * h + k * c
            out_ref[pl.ds(row, c), :] = x_ref[pl.ds(mx * h + k * c, c), :].astype(
                out_ref.dtype
            )
            s = copy(row, y_send_sems.at[k], y_recv_sems.at[k], yp)
            s.start()
            y_sends.append(s)

        out_ref[pl.ds(my * m + (1 - mx) * h, h), :] = x_ref[
            pl.ds((1 - mx) * h, h), :
        ].astype(out_ref.dtype)

        x_sends = []
        for k in range(K):
            row = (1 - my) * m + mx * h + k * c
            copy(row, y_send_sems.at[k], y_recv_sems.at[k], yp).wait_recv()
            f = copy(row, x_send_sems.at[k], x_recv_sems.at[k], xp)
            f.start()
            x_sends.append(f)

        for k in range(K):
            row = (1 - my) * m + (1 - mx) * h + k * c
            copy(row, x_send_sems.at[k], x_recv_sems.at[k], xp).wait_recv()

        for s in y_sends:
            s.wait_send()
        for s in x_sends:
            s.wait_send()

    out_shape = jax.ShapeDtypeStruct((2 * m, n), jnp.bfloat16)
    return pl.pallas_call(
        body,
        out_shape=out_shape,
        in_specs=[pl.BlockSpec(memory_space=pltpu.VMEM)],
        out_specs=pl.BlockSpec(memory_space=pltpu.VMEM),
        scratch_shapes=[
            pltpu.SemaphoreType.DMA((K,)),
            pltpu.SemaphoreType.DMA((K,)),
            pltpu.SemaphoreType.DMA((K,)),
            pltpu.SemaphoreType.DMA((K,)),
        ],
        compiler_params=pltpu.CompilerParams(collective_id=0),
    )(x)


# device time: 63466 ns/iter; 1.6673x vs baseline; 1.0362x over previous
import jax
import jax.numpy as jnp
from jax import lax
from jax.experimental import pallas as pl
from jax.experimental.pallas import tpu as pltpu

K = 16
L = 4


def kernel(x):
    m, n = x.shape
    h = m // 2
    c = h // K

    def body(
        x_ref,
        out_ref,
        f32_s,
        f32_k,
        send_buf,
        yrecv_buf,
        keep_buf,
        load_sems,
        store_sem,
        y_send_sems,
        y_recv_sems,
        x_send_sems,
        x_recv_sems,
    ):
        mx = lax.axis_index("x")
        my = lax.axis_index("y")
        yp = (mx, 1 - my)
        xp = (1 - mx, my)

        load_s = pltpu.make_async_copy(
            x_ref.at[pl.ds(mx * h, h), :], f32_s, load_sems.at[0]
        )
        load_k = pltpu.make_async_copy(
            x_ref.at[pl.ds((1 - mx) * h, h), :], f32_k, load_sems.at[1]
        )
        load_s.start()
        load_k.start()

        barrier = pltpu.get_barrier_semaphore()
        for nbr in (yp, xp):
            pl.semaphore_signal(
                barrier, inc=1, device_id=nbr, device_id_type=pl.DeviceIdType.MESH
            )
        pl.semaphore_wait(barrier, 2)

        load_s.wait()

        y_sends = []
        x_fwds = []
        stores = []

        def cast_and_send(k):
            ds = pl.ds(k * c, c)
            send_buf[ds, :] = f32_s[ds, :].astype(send_buf.dtype)
            s = pltpu.make_async_remote_copy(
                src_ref=send_buf.at[ds, :],
                dst_ref=yrecv_buf.at[ds, :],
                send_sem=y_send_sems.at[k],
                recv_sem=y_recv_sems.at[k],
                device_id=yp,
                device_id_type=pl.DeviceIdType.MESH,
            )
            s.start()
            y_sends.append(s)
            st = pltpu.make_async_copy(
                send_buf.at[ds, :],
                out_ref.at[pl.ds(my * m + mx * h + k * c, c), :],
                store_sem,
            )
            st.start()
            stores.append(st)

        for k in range(L):
            cast_and_send(k)

        load_k.wait()

        for k in range(K):
            if k + L < K:
                cast_and_send(k + L)
            ds = pl.ds(k * c, c)
            pltpu.make_async_remote_copy(
                src_ref=yrecv_buf.at[ds, :],
                dst_ref=yrecv_buf.at[ds, :],
                send_sem=y_send_sems.at[k],
                recv_sem=y_recv_sems.at[k],
                device_id=yp,
                device_id_type=pl.DeviceIdType.MESH,
            ).wait_recv()
            grow = (1 - my) * m + mx * h + k * c
            f = pltpu.make_async_remote_copy(
                src_ref=yrecv_buf.at[ds, :],
                dst_ref=out_ref.at[pl.ds(grow, c), :],
                send_sem=x_send_sems.at[k],
                recv_sem=x_recv_sems.at[k],
                device_id=xp,
                device_id_type=pl.DeviceIdType.MESH,
            )
            f.start()
            x_fwds.append(f)
            st = pltpu.make_async_copy(
                yrecv_buf.at[ds, :], out_ref.at[pl.ds(grow, c), :], store_sem
            )
            st.start()
            stores.append(st)
            keep_buf[ds, :] = f32_k[ds, :].astype(keep_buf.dtype)
            st2 = pltpu.make_async_copy(
                keep_buf.at[ds, :],
                out_ref.at[pl.ds(my * m + (1 - mx) * h + k * c, c), :],
                store_sem,
            )
            st2.start()
            stores.append(st2)

        for k in range(K):
            xr = pl.ds((1 - my) * m + (1 - mx) * h + k * c, c)
            pltpu.make_async_remote_copy(
                src_ref=out_ref.at[xr, :],
                dst_ref=out_ref.at[xr, :],
                send_sem=x_send_sems.at[k],
                recv_sem=x_recv_sems.at[k],
                device_id=xp,
                device_id_type=pl.DeviceIdType.MESH,
            ).wait_recv()

        for s in y_sends:
            s.wait_send()
        for f in x_fwds:
            f.wait_send()
        for st in stores:
            st.wait()

    out_shape = jax.ShapeDtypeStruct((2 * m, n), jnp.bfloat16)
    return pl.pallas_call(
        body,
        out_shape=out_shape,
        in_specs=[pl.BlockSpec(memory_space=pl.ANY)],
        out_specs=pl.BlockSpec(memory_space=pl.ANY),
        scratch_shapes=[
            pltpu.VMEM((h, n), jnp.float32),
            pltpu.VMEM((h, n), jnp.float32),
            pltpu.VMEM((h, n), jnp.bfloat16),
            pltpu.VMEM((h, n), jnp.bfloat16),
            pltpu.VMEM((h, n), jnp.bfloat16),
            pltpu.SemaphoreType.DMA((2,)),
            pltpu.SemaphoreType.DMA,
            pltpu.SemaphoreType.DMA((K,)),
            pltpu.SemaphoreType.DMA((K,)),
            pltpu.SemaphoreType.DMA((K,)),
            pltpu.SemaphoreType.DMA((K,)),
        ],
        compiler_params=pltpu.CompilerParams(collective_id=0),
    )(x)


# device time: 61159 ns/iter; 1.7302x vs baseline; 1.0377x over previous
import jax
import jax.numpy as jnp
from jax import lax
from jax.experimental import pallas as pl
from jax.experimental.pallas import tpu as pltpu

K = 32
Q = 8


def kernel(x):
    m, n = x.shape
    h = m // 2
    c = h // K
    b = c * Q

    def body(
        x_ref,
        out_ref,
        f32_s,
        f32_k,
        send_buf,
        yrecv_buf,
        keep_buf,
        load_sems,
        keep_load_sem,
        store_sem,
        y_send_sems,
        y_recv_sems,
        x_send_sems,
        x_recv_sems,
    ):
        nq = K // Q
        mx = lax.axis_index("x")
        my = lax.axis_index("y")
        yp = (mx, 1 - my)
        xp = (1 - mx, my)

        qloads = []
        for q in range(nq):
            ld = pltpu.make_async_copy(
                x_ref.at[pl.ds(mx * h + q * b, b), :],
                f32_s.at[pl.ds(q * b, b), :],
                load_sems.at[q],
            )
            ld.start()
            qloads.append(ld)
        load_k = pltpu.make_async_copy(
            x_ref.at[pl.ds((1 - mx) * h, h), :], f32_k, keep_load_sem
        )
        load_k.start()

        barrier = pltpu.get_barrier_semaphore()
        for nbr in (yp, xp):
            pl.semaphore_signal(
                barrier, inc=1, device_id=nbr, device_id_type=pl.DeviceIdType.MESH
            )
        pl.semaphore_wait(barrier, 2)

        y_sends = []
        x_fwds = []
        stores = []

        for q in range(nq):
            qloads[q].wait()
            for k in range(q * Q, (q + 1) * Q):
                ds = pl.ds(k * c, c)
                send_buf[ds, :] = f32_s[ds, :].astype(send_buf.dtype)
                s = pltpu.make_async_remote_copy(
                    src_ref=send_buf.at[ds, :],
                    dst_ref=yrecv_buf.at[ds, :],
                    send_sem=y_send_sems.at[k],
                    recv_sem=y_recv_sems.at[k],
                    device_id=yp,
                    device_id_type=pl.DeviceIdType.MESH,
                )
                s.start()
                y_sends.append(s)
            st = pltpu.make_async_copy(
                send_buf.at[pl.ds(q * b, b), :],
                out_ref.at[pl.ds(my * m + mx * h + q * b, b), :],
                store_sem,
            )
            st.start()
            stores.append(st)

        for k in range(K):
            ds = pl.ds(k * c, c)
            pltpu.make_async_remote_copy(
                src_ref=yrecv_buf.at[ds, :],
                dst_ref=yrecv_buf.at[ds, :],
                send_sem=y_send_sems.at[k],
                recv_sem=y_recv_sems.at[k],
                device_id=yp,
                device_id_type=pl.DeviceIdType.MESH,
            ).wait_recv()
            grow = (1 - my) * m + mx * h + k * c
            f = pltpu.make_async_remote_copy(
                src_ref=yrecv_buf.at[ds, :],
                dst_ref=out_ref.at[pl.ds(grow, c), :],
                send_sem=x_send_sems.at[k],
                recv_sem=x_recv_sems.at[k],
                device_id=xp,
                device_id_type=pl.DeviceIdType.MESH,
            )
            f.start()
            x_fwds.append(f)
            if k % Q == Q - 1:
                q = k // Q
                st = pltpu.make_async_copy(
                    yrecv_buf.at[pl.ds(q * b, b), :],
                    out_ref.at[pl.ds((1 - my) * m + mx * h + q * b, b), :],
                    store_sem,
                )
                st.start()
                stores.append(st)

        load_k.wait()
        keep_buf[...] = f32_k[...].astype(keep_buf.dtype)
        st = pltpu.make_async_copy(
            keep_buf,
            out_ref.at[pl.ds(my * m + (1 - mx) * h, h), :],
            store_sem,
        )
        st.start()
        stores.append(st)

        for k in range(K):
            xr = pl.ds((1 - my) * m + (1 - mx) * h + k * c, c)
            pltpu.make_async_remote_copy(
                src_ref=out_ref.at[xr, :],
                dst_ref=out_ref.at[xr, :],
                send_sem=x_send_sems.at[k],
                recv_sem=x_recv_sems.at[k],
                device_id=xp,
                device_id_type=pl.DeviceIdType.MESH,
            ).wait_recv()

        for s in y_sends:
            s.wait_send()
        for f in x_fwds:
            f.wait_send()
        for st in stores:
            st.wait()

    out_shape = jax.ShapeDtypeStruct((2 * m, n), jnp.bfloat16)
    return pl.pallas_call(
        body,
        out_shape=out_shape,
        in_specs=[pl.BlockSpec(memory_space=pl.ANY)],
        out_specs=pl.BlockSpec(memory_space=pl.ANY),
        scratch_shapes=[
            pltpu.VMEM((h, n), jnp.float32),
            pltpu.VMEM((h, n), jnp.float32),
            pltpu.VMEM((h, n), jnp.bfloat16),
            pltpu.VMEM((h, n), jnp.bfloat16),
            pltpu.VMEM((h, n), jnp.bfloat16),
            pltpu.SemaphoreType.DMA((K // Q,)),
            pltpu.SemaphoreType.DMA,
            pltpu.SemaphoreType.DMA,
            pltpu.SemaphoreType.DMA((K,)),
            pltpu.SemaphoreType.DMA((K,)),
            pltpu.SemaphoreType.DMA((K,)),
            pltpu.SemaphoreType.DMA((K,)),
        ],
        compiler_params=pltpu.CompilerParams(collective_id=0),
    )(x)
